# baseline (speedup 1.0000x reference)
import functools

import jax
import jax.numpy as jnp
from jax import lax
from jax.experimental import pallas as pl
from jax.experimental.pallas import tpu as pltpu
from jax.experimental.pallas import tpu_sc as plsc

_VOCAB = 100000
_D = 64
_B = 16384
_NEG = 3
_NEG_PAD = 8
_L = 16

_NC, _NS = 2, 16
_NW = _NC * _NS
_BPW = _B // _NW
_NGRP = _BPW // _L


@functools.cache
def _build_sc_dots():
    mesh = plsc.VectorSubcoreMesh(core_axis_name="c", subcore_axis_name="s")

    @functools.partial(
        pl.kernel,
        mesh=mesh,
        compiler_params=pltpu.CompilerParams(
            use_tc_tiling_on_sc=False, needs_layout_passes=False),
        out_type=jax.ShapeDtypeStruct((_NW * 4, _BPW), jnp.float32),
        scratch_types=[
            pltpu.VMEM((_BPW,), jnp.int32),
            pltpu.VMEM((_BPW, _D), jnp.float32),
            pltpu.VMEM((_BPW,), jnp.int32),
            pltpu.VMEM((_BPW, _D), jnp.float32),
            pltpu.VMEM((_NEG_PAD,), jnp.int32),
            pltpu.VMEM((_NEG_PAD, _D), jnp.float32),
            pltpu.VMEM((4, _BPW), jnp.float32),
            pltpu.SemaphoreType.DMA,
            pltpu.SemaphoreType.DMA,
            pltpu.SemaphoreType.DMA,
        ],
    )
    def _sc_dots(emb_hbm, tidx_hbm, cidx_hbm, nidx_hbm, dots_out,
                 tiv, trv, civ, crv, niv, nrv, outv, sem_t, sem_c, sem_n):
        wid = lax.axis_index("s") * _NC + lax.axis_index("c")
        base = wid * _BPW
        pltpu.sync_copy(tidx_hbm.at[pl.ds(base, _BPW)], tiv)
        pltpu.sync_copy(cidx_hbm.at[pl.ds(base, _BPW)], civ)
        pltpu.sync_copy(nidx_hbm, niv)
        cp_t = pltpu.async_copy(emb_hbm.at[tiv], trv, sem_t)
        cp_c = pltpu.async_copy(emb_hbm.at[civ], crv, sem_c)
        cp_n = pltpu.async_copy(emb_hbm.at[niv], nrv, sem_n)
        cp_t.wait()
        cp_c.wait()
        cp_n.wait()

        nv = [[nrv[i, pl.ds(k * _L, _L)] for k in range(_D // _L)]
              for i in range(_NEG)]

        def group(g, _):
            rows = g * _L + lax.iota(jnp.int32, _L)
            accp = jnp.zeros((_L,), jnp.float32)
            acc0 = jnp.zeros((_L,), jnp.float32)
            acc1 = jnp.zeros((_L,), jnp.float32)
            acc2 = jnp.zeros((_L,), jnp.float32)
            for d in range(_D):
                dv = jnp.full((_L,), d, jnp.int32)
                tcol = plsc.load_gather(trv, [rows, dv])
                ccol = plsc.load_gather(crv, [rows, dv])
                k, l = d // _L, d % _L
                accp = accp + tcol * ccol
                acc0 = acc0 + tcol * jnp.broadcast_to(nv[0][k][l], (_L,))
                acc1 = acc1 + tcol * jnp.broadcast_to(nv[1][k][l], (_L,))
                acc2 = acc2 + tcol * jnp.broadcast_to(nv[2][k][l], (_L,))
            outv[0, pl.ds(g * _L, _L)] = accp
            outv[1, pl.ds(g * _L, _L)] = acc0
            outv[2, pl.ds(g * _L, _L)] = acc1
            outv[3, pl.ds(g * _L, _L)] = acc2
            return _

        lax.fori_loop(0, _NGRP, group, None)
        pltpu.sync_copy(outv, dots_out.at[pl.ds(wid * 4, 4)])

    return _sc_dots


def _tc_body(dots_ref, out_ref):
    x = dots_ref[...]
    pos_row = lax.broadcasted_iota(jnp.int32, (_NW * 4, _BPW), 0) % 4 == 0
    s = jnp.where(pos_row, x, -x)
    ls = jnp.minimum(s, 0.0) - jnp.log1p(jnp.exp(-jnp.abs(s)))
    out_ref[0, 0] = jnp.sum(ls)


def _tc_loss(dots):
    return pl.pallas_call(
        _tc_body,
        out_specs=pl.BlockSpec(memory_space=pltpu.SMEM),
        out_shape=jax.ShapeDtypeStruct((1, 1), jnp.float32),
    )(dots)


def kernel(target_idx, context_idx, embeddings, neg_idx):
    nidx = jnp.concatenate(
        [neg_idx.astype(jnp.int32),
         jnp.zeros((_NEG_PAD - _NEG,), jnp.int32)])
    dots = _build_sc_dots()(
        embeddings, target_idx.astype(jnp.int32), context_idx.astype(jnp.int32),
        nidx)
    acc = _tc_loss(dots)
    return -acc[0, 0] / _B


# speedup vs baseline: 1.3863x; 1.3863x over previous
import functools

import jax
import jax.numpy as jnp
from jax import lax
from jax.experimental import pallas as pl
from jax.experimental.pallas import tpu as pltpu
from jax.experimental.pallas import tpu_sc as plsc

_VOCAB = 100000
_D = 64
_B = 16384
_NEG = 3
_L = 16

_NC, _NS = 2, 16
_NW = _NC * _NS
_NROUND = _D // _NW
_NQ = 4
_QB = _B // _NQ
_ROWLEN = _NQ * _QB * 4


@functools.cache
def _build_sc_dim_dots():
    mesh = plsc.VectorSubcoreMesh(core_axis_name="c", subcore_axis_name="s")

    @functools.partial(
        pl.kernel,
        mesh=mesh,
        compiler_params=pltpu.CompilerParams(
            use_tc_tiling_on_sc=True, needs_layout_passes=False),
        out_type=jax.ShapeDtypeStruct((_D, _ROWLEN), jnp.float32),
        scratch_types=[
            pltpu.VMEM((_VOCAB,), jnp.float32),
            pltpu.VMEM((_QB,), jnp.int32),
            pltpu.VMEM((_QB,), jnp.int32),
            pltpu.VMEM((_L,), jnp.int32),
            pltpu.VMEM((4 * _QB,), jnp.float32),
            pltpu.SemaphoreType.DMA,
        ],
    )
    def _sc_dim_dots(embT_hbm, tidx_hbm, cidx_hbm, nidx_hbm, parts_out,
                     ed, tqi, cqi, niv, accv, sem):
        wid = lax.axis_index("s") * _NC + lax.axis_index("c")
        pltpu.sync_copy(nidx_hbm, niv)
        nvals = niv[...]

        def do_round(r, _):
            d = r * _NW + wid
            pltpu.sync_copy(embT_hbm.at[d], ed)
            nd = plsc.load_gather(ed, [nvals])
            nb0 = jnp.broadcast_to(nd[0], (_L,))
            nb1 = jnp.broadcast_to(nd[1], (_L,))
            nb2 = jnp.broadcast_to(nd[2], (_L,))
            row = r * _NW + wid

            def do_quarter(q, _):
                pltpu.sync_copy(tidx_hbm.at[pl.ds(q * _QB, _QB)], tqi)
                pltpu.sync_copy(cidx_hbm.at[pl.ds(q * _QB, _QB)], cqi)

                def group(j, _):
                    tv = tqi[pl.ds(j * _L, _L)]
                    cv = cqi[pl.ds(j * _L, _L)]
                    tcol = plsc.load_gather(ed, [tv])
                    ccol = plsc.load_gather(ed, [cv])
                    accv[pl.ds(j * _L, _L)] = tcol * ccol
                    accv[pl.ds(_QB + j * _L, _L)] = tcol * nb0
                    accv[pl.ds(2 * _QB + j * _L, _L)] = tcol * nb1
                    accv[pl.ds(3 * _QB + j * _L, _L)] = tcol * nb2
                    return _

                lax.fori_loop(0, _QB // _L, group, None)
                pltpu.sync_copy(
                    accv, parts_out.at[row, pl.ds(q * 4 * _QB, 4 * _QB)])
                return _

            lax.fori_loop(0, _NQ, do_quarter, None)
            return _

        lax.fori_loop(0, _NROUND, do_round, None)

    return _sc_dim_dots


_TCBLK = 4096


def _tc_body(parts_ref, out_ref):
    i = pl.program_id(0)
    k = pl.program_id(1)
    x = parts_ref[...]
    s = jnp.sum(x, axis=0, keepdims=True)
    s = jnp.where(k == 0, s, -s)
    ls = jnp.minimum(s, 0.0) - jnp.log1p(jnp.exp(-jnp.abs(s)))
    total = jnp.sum(ls)

    @pl.when(jnp.logical_and(i == 0, k == 0))
    def _():
        out_ref[0, 0] = 0.0

    out_ref[0, 0] += total


def _tc_loss(parts):
    return pl.pallas_call(
        _tc_body,
        grid=(_NQ, 4),
        in_specs=[
            pl.BlockSpec((_D, _TCBLK), lambda i, k: (0, i * 4 + k)),
        ],
        out_specs=pl.BlockSpec(memory_space=pltpu.SMEM),
        out_shape=jax.ShapeDtypeStruct((1, 1), jnp.float32),
    )(parts)


def kernel(target_idx, context_idx, embeddings, neg_idx):
    nidx = jnp.concatenate(
        [neg_idx.astype(jnp.int32), jnp.zeros((_L - _NEG,), jnp.int32)])
    parts = _build_sc_dim_dots()(
        embeddings.T, target_idx.astype(jnp.int32),
        context_idx.astype(jnp.int32), nidx)
    acc = _tc_loss(parts)
    return -acc[0, 0] / _B
